# Initial kernel scaffold; baseline (speedup 1.0000x reference)
import jax
import jax.numpy as jnp
from jax.experimental import pallas as pl


def kernel(features, nodes, neigh_idx, weight):
    raise NotImplementedError("write your pallas kernel here")


# speedup vs baseline: 1.2592x; 1.2592x over previous
import functools

import jax
import jax.numpy as jnp
from jax import lax
from jax.experimental import pallas as pl
from jax.experimental.pallas import tpu as pltpu
from jax.experimental.pallas import tpu_sc as plsc

NC = 2
NS = 16
L = 16
NW = NC * NS

B = 1024
S = 16
D = 128
E = 128

B_PER_W = B // NW
ROWS_PER_W = B_PER_W * S
GW = 128
N_CH = ROWS_PER_W // GW


def _sc_gather_mean(features, nodes, neigh_flat):
    mesh = plsc.VectorSubcoreMesh(core_axis_name="c", subcore_axis_name="s")

    @functools.partial(
        pl.kernel,
        out_type=(
            jax.ShapeDtypeStruct((B, D), jnp.float32),
            jax.ShapeDtypeStruct((B, D), jnp.float32),
        ),
        mesh=mesh,
        scratch_types=[
            pltpu.VMEM((ROWS_PER_W,), jnp.int32),
            pltpu.VMEM((B_PER_W,), jnp.int32),
            pltpu.VMEM((ROWS_PER_W, D), jnp.float32),
            pltpu.VMEM((B_PER_W, D), jnp.float32),
            pltpu.VMEM((B_PER_W, D), jnp.float32),
            pltpu.SemaphoreType.DMA,
            pltpu.SemaphoreType.DMA,
        ],
    )
    def k(feat_hbm, nodes_hbm, nidx_hbm, self_out, mean_out,
          nidx_v, sidx_v, rows_v, self_v, mean_v, sem, sem2):
        wid = lax.axis_index("s") * NC + lax.axis_index("c")
        base = wid * B_PER_W
        rbase = wid * ROWS_PER_W

        pltpu.sync_copy(nidx_hbm.at[pl.ds(rbase, ROWS_PER_W)], nidx_v)
        pltpu.sync_copy(nodes_hbm.at[pl.ds(base, B_PER_W)], sidx_v)

        copies = []
        for j in range(N_CH):
            copies.append(pltpu.async_copy(
                feat_hbm.at[nidx_v.at[pl.ds(j * GW, GW)]],
                rows_v.at[pl.ds(j * GW, GW)], sem))
        self_copy = pltpu.async_copy(feat_hbm.at[sidx_v], self_v, sem2)
        for c in copies:
            c.wait()

        @pl.loop(0, B_PER_W)
        def _(n):
            row0 = n * S
            for c in range(D // L):
                cs = pl.ds(c * L, L)
                acc = rows_v[row0, cs]
                for r in range(1, S):
                    acc = acc + rows_v[row0 + r, cs]
                mean_v[n, cs] = acc * (1.0 / S)

        self_copy.wait()
        pltpu.sync_copy(self_v, self_out.at[pl.ds(base, B_PER_W)])
        pltpu.sync_copy(mean_v, mean_out.at[pl.ds(base, B_PER_W)])

    return k(features, nodes, neigh_flat)


def _tc_bmm(selff, meanf, weight):
    Bb = 64

    def body(s_ref, m_ref, w_ref, o_ref):
        w = w_ref[...]
        s = s_ref[...]
        m = m_ref[...]
        acc = jnp.sum(s[:, :, None] * w[:, :D, :], axis=1)
        acc = acc + jnp.sum(m[:, :, None] * w[:, D:, :], axis=1)
        o_ref[...] = jnp.maximum(acc, 0.0)

    return pl.pallas_call(
        body,
        grid=(B // Bb,),
        in_specs=[
            pl.BlockSpec((Bb, D), lambda i: (i, 0)),
            pl.BlockSpec((Bb, D), lambda i: (i, 0)),
            pl.BlockSpec((Bb, 2 * D, E), lambda i: (i, 0, 0)),
        ],
        out_specs=pl.BlockSpec((Bb, E), lambda i: (i, 0)),
        out_shape=jax.ShapeDtypeStruct((B, E), jnp.float32),
    )(selff, meanf, weight)


def kernel(features, nodes, neigh_idx, weight):
    nodes = nodes.astype(jnp.int32)
    neigh_flat = neigh_idx.astype(jnp.int32).reshape(-1)
    selff, meanf = _sc_gather_mean(features, nodes, neigh_flat)
    return _tc_bmm(selff, meanf, weight)


# speedup vs baseline: 1.3948x; 1.1077x over previous
import functools

import jax
import jax.numpy as jnp
from jax import lax
from jax.experimental import pallas as pl
from jax.experimental.pallas import tpu as pltpu
from jax.experimental.pallas import tpu_sc as plsc

NC = 2
NS = 16
L = 16
NW = NC * NS

B = 1024
S = 16
D = 128
E = 128

B_PER_W = B // NW
ROWS_PER_W = B_PER_W * S
GW = 128
N_CH = ROWS_PER_W // GW


def _sc_gather_mean(features, nodes, neigh_flat):
    mesh = plsc.VectorSubcoreMesh(core_axis_name="c", subcore_axis_name="s")

    @functools.partial(
        pl.kernel,
        out_type=(
            jax.ShapeDtypeStruct((B, D), jnp.float32),
            jax.ShapeDtypeStruct((B, D), jnp.float32),
        ),
        mesh=mesh,
        scratch_types=[
            pltpu.VMEM((ROWS_PER_W,), jnp.int32),
            pltpu.VMEM((B_PER_W,), jnp.int32),
            pltpu.VMEM((ROWS_PER_W, D), jnp.float32),
            pltpu.VMEM((B_PER_W, D), jnp.float32),
            pltpu.VMEM((B_PER_W, D), jnp.float32),
            pltpu.SemaphoreType.DMA,
            pltpu.SemaphoreType.DMA,
        ],
    )
    def k(feat_hbm, nodes_hbm, nidx_hbm, self_out, mean_out,
          nidx_v, sidx_v, rows_v, self_v, mean_v, sem, sem2):
        wid = lax.axis_index("s") * NC + lax.axis_index("c")
        base = wid * B_PER_W
        rbase = wid * ROWS_PER_W

        pltpu.sync_copy(nidx_hbm.at[pl.ds(rbase, ROWS_PER_W)], nidx_v)
        pltpu.sync_copy(nodes_hbm.at[pl.ds(base, B_PER_W)], sidx_v)

        copies = []
        for j in range(N_CH):
            copies.append(pltpu.async_copy(
                feat_hbm.at[nidx_v.at[pl.ds(j * GW, GW)]],
                rows_v.at[pl.ds(j * GW, GW)], sem))
        self_copy = pltpu.async_copy(feat_hbm.at[sidx_v], self_v, sem2)
        for c in copies:
            c.wait()

        @pl.loop(0, B_PER_W)
        def _(n):
            row0 = n * S
            for c in range(D // L):
                cs = pl.ds(c * L, L)
                acc = rows_v[row0, cs]
                for r in range(1, S):
                    acc = acc + rows_v[row0 + r, cs]
                mean_v[n, cs] = acc * (1.0 / S)

        self_copy.wait()
        pltpu.sync_copy(self_v, self_out.at[pl.ds(base, B_PER_W)])
        pltpu.sync_copy(mean_v, mean_out.at[pl.ds(base, B_PER_W)])

    return k(features, nodes, neigh_flat)


def _tc_bmm(selff, meanf, weight):
    Bb = 64

    def body(s_ref, m_ref, w_ref, o_ref):
        s = s_ref[...]
        m = m_ref[...]
        c = jnp.concatenate([s, m], axis=1)
        acc = jax.lax.dot_general(
            c, w_ref[...],
            dimension_numbers=(((1,), (1,)), ((0,), (0,))),
            preferred_element_type=jnp.float32)
        o_ref[...] = jnp.maximum(acc, 0.0)

    return pl.pallas_call(
        body,
        grid=(B // Bb,),
        in_specs=[
            pl.BlockSpec((Bb, D), lambda i: (i, 0)),
            pl.BlockSpec((Bb, D), lambda i: (i, 0)),
            pl.BlockSpec((Bb, 2 * D, E), lambda i: (i, 0, 0)),
        ],
        out_specs=pl.BlockSpec((Bb, E), lambda i: (i, 0)),
        out_shape=jax.ShapeDtypeStruct((B, E), jnp.float32),
    )(selff, meanf, weight)


def kernel(features, nodes, neigh_idx, weight):
    nodes = nodes.astype(jnp.int32)
    neigh_flat = neigh_idx.astype(jnp.int32).reshape(-1)
    selff, meanf = _sc_gather_mean(features, nodes, neigh_flat)
    return _tc_bmm(selff, meanf, weight)
